# baseline (speedup 1.0000x reference)
import functools

import jax
import jax.numpy as jnp
from jax import lax
from jax.experimental import pallas as pl
from jax.experimental.pallas import tpu as pltpu
from jax.experimental.pallas import tpu_sc as plsc

EMBED_DIM = 32
ROWS_PER_MEGA = 128 // EMBED_DIM
NUM_CORES = 2
NUM_SUBCORES = 16
NUM_WORKERS = NUM_CORES * NUM_SUBCORES
CHUNK = 128
STRIP = 512


@functools.lru_cache(maxsize=None)
def _make_kernel(n_h: int, n_b: int, vocab: int):
    n_mega = vocab // ROWS_PER_MEGA
    n_strips = -(-vocab // STRIP)
    strips_per_w = -(-n_strips // NUM_WORKERS)
    mesh = plsc.VectorSubcoreMesh(
        core_axis_name="c",
        subcore_axis_name="s",
        num_cores=NUM_CORES,
        num_subcores=NUM_SUBCORES,
    )

    @functools.partial(
        pl.kernel,
        out_type=(
            jax.ShapeDtypeStruct((n_h, EMBED_DIM, n_b), jnp.float32),
            jax.ShapeDtypeStruct((n_mega, 128), jnp.float32),
        ),
        mesh=mesh,
        scratch_types=[
            pltpu.VMEM((EMBED_DIM, STRIP), jnp.float32),
            pltpu.VMEM((STRIP // ROWS_PER_MEGA, 128), jnp.float32),
            pltpu.VMEM((n_h, CHUNK), jnp.int32),
            pltpu.VMEM((2, CHUNK), jnp.int32),
            pltpu.VMEM((CHUNK,), jnp.int32),
            pltpu.VMEM((2, CHUNK, 128), jnp.float32),
            pltpu.VMEM((EMBED_DIM, CHUNK), jnp.float32),
            pltpu.SemaphoreType.DMA,
            pltpu.SemaphoreType.DMA,
            pltpu.SemaphoreType.REGULAR,
        ],
        compiler_params=pltpu.CompilerParams(needs_layout_passes=False),
    )
    def body(idx_hbm, table_hbm, tail_hbm, out_hbm, tscr_hbm, tin_v, tout_v,
             idx_v, mega_v, rem_v, rows_v, outt_v, gsem, ssem, bsem):
        wid = lax.axis_index("s") * NUM_CORES + lax.axis_index("c")
        b0 = wid * CHUNK
        pltpu.sync_copy(idx_hbm.at[:, pl.ds(b0, CHUNK)], idx_v)

        shr2 = lax.iota(jnp.int32, 16) >> 2
        n_full = vocab // STRIP
        tail = vocab - n_full * STRIP

        def transpose_strip(v0, nv):
            def j_body(j, carry2):
                rowv = shr2 + 4 * j
                for d in range(EMBED_DIM):
                    colv = (lax.iota(jnp.int32, 16) & 3) * EMBED_DIM + d
                    vals = tin_v[d, pl.ds(j * 16, 16)]
                    plsc.store_scatter(tout_v, [rowv, colv], vals)
                return carry2

            lax.fori_loop(0, nv // 16, j_body, 0)
            m0 = pl.multiple_of(v0 >> 2, STRIP // ROWS_PER_MEGA)
            pltpu.sync_copy(
                tout_v.at[pl.ds(0, nv // ROWS_PER_MEGA)],
                tscr_hbm.at[pl.ds(m0, nv // ROWS_PER_MEGA)],
            )

        def strip_body(k, carry):
            st = wid * strips_per_w + k

            @pl.when(st < n_full)
            def _():
                v0 = pl.multiple_of(st * STRIP, STRIP)
                pltpu.async_copy(
                    table_hbm.at[:, pl.ds(v0, STRIP)], tin_v, ssem
                ).wait()
                transpose_strip(v0, STRIP)

            return carry

        lax.fori_loop(0, strips_per_w, strip_body, 0)
        if tail:
            @pl.when(wid == NUM_WORKERS - 1)
            def _():
                pltpu.sync_copy(
                    tail_hbm,
                    tscr_hbm.at[
                        pl.ds(n_full * (STRIP // ROWS_PER_MEGA),
                              tail // ROWS_PER_MEGA)
                    ],
                )
        plsc.subcore_barrier()
        pltpu.core_barrier(bsem, core_axis_name="c")
        plsc.subcore_barrier()

        def fire(h, buf):
            for g in range(CHUNK // 16):
                sl = pl.ds(g * 16, 16)
                mega_v[buf, sl] = lax.shift_right_logical(idx_v[h, sl], 2)
            return pltpu.async_copy(
                tscr_hbm.at[mega_v.at[buf]], rows_v.at[buf], gsem
            )

        fire(0, 0).wait()

        def chunk_body(h, carry):
            buf = lax.rem(h, 2)
            @pl.when(h + 1 < n_h)
            def _():
                fire(h + 1, 1 - buf)

            for g in range(CHUNK // 16):
                sl = pl.ds(g * 16, 16)
                rem_v[sl] = (idx_v[h, sl] & 3) * EMBED_DIM

            for g in range(CHUNK // 16):
                sl = pl.ds(g * 16, 16)
                rowv = lax.iota(jnp.int32, 16) + g * 16
                remg = rem_v[sl]
                for d in range(EMBED_DIM):
                    outt_v[d, sl] = plsc.load_gather(
                        rows_v.at[buf], [rowv, remg + d]
                    )

            pltpu.sync_copy(outt_v, out_hbm.at[h, :, pl.ds(b0, CHUNK)])

            @pl.when(h + 1 < n_h)
            def _():
                pltpu.make_async_copy(
                    tscr_hbm.at[mega_v.at[1 - buf]], rows_v.at[1 - buf], gsem
                ).wait()

            return carry

        lax.fori_loop(0, n_h, chunk_body, 0)

    return body


def kernel(inputs, table):
    b, h = inputs.shape
    vocab = table.shape[0]
    idx_t = inputs.astype(jnp.int32).T
    table_t = table.T
    n_full = vocab // STRIP
    tail = vocab - n_full * STRIP
    if tail:
        tail_mega = table[n_full * STRIP:].reshape(tail // ROWS_PER_MEGA, 128)
    else:
        tail_mega = jnp.zeros((8, 128), jnp.float32)
    out_phys, _ = _make_kernel(h, b, vocab)(idx_t, table_t, tail_mega)
    return jnp.transpose(out_phys, (2, 0, 1))


# speedup vs baseline: 2.4328x; 2.4328x over previous
import functools

import jax
import jax.numpy as jnp
from jax import lax
from jax.experimental import pallas as pl
from jax.experimental.pallas import tpu as pltpu
from jax.experimental.pallas import tpu_sc as plsc

EMBED_DIM = 32
ROWS_PER_MEGA = 128 // EMBED_DIM
NUM_CORES = 2
NUM_SUBCORES = 16
NUM_WORKERS = NUM_CORES * NUM_SUBCORES
CHUNK = 128
STRIP = 512


@functools.lru_cache(maxsize=None)
def _make_kernel(n_h: int, n_b: int, vocab: int):
    n_mega = vocab // ROWS_PER_MEGA
    n_strips = -(-vocab // STRIP)
    strips_per_w = -(-n_strips // NUM_WORKERS)
    mesh = plsc.VectorSubcoreMesh(
        core_axis_name="c",
        subcore_axis_name="s",
        num_cores=NUM_CORES,
        num_subcores=NUM_SUBCORES,
    )

    @functools.partial(
        pl.kernel,
        out_type=(
            jax.ShapeDtypeStruct((n_h, EMBED_DIM, n_b), jnp.float32),
            jax.ShapeDtypeStruct((n_mega, 128), jnp.float32),
        ),
        mesh=mesh,
        scratch_types=[
            pltpu.VMEM((2, EMBED_DIM, STRIP), jnp.float32),
            pltpu.VMEM((STRIP // ROWS_PER_MEGA, 128), jnp.float32),
            pltpu.VMEM((n_h, CHUNK), jnp.int32),
            pltpu.VMEM((2, CHUNK), jnp.int32),
            pltpu.VMEM((CHUNK,), jnp.int32),
            pltpu.VMEM((2, CHUNK, 128), jnp.float32),
            pltpu.VMEM((2, EMBED_DIM, CHUNK), jnp.float32),
            pltpu.SemaphoreType.DMA,
            pltpu.SemaphoreType.DMA,
            pltpu.SemaphoreType.DMA,
            pltpu.SemaphoreType.REGULAR,
        ],
        compiler_params=pltpu.CompilerParams(needs_layout_passes=False),
    )
    def body(idx_hbm, table_hbm, tail_hbm, out_hbm, tscr_hbm, tin_v, tout_v,
             idx_v, mega_v, rem_v, rows_v, outt_v, gsem, ssem, osem, bsem):
        wid = lax.axis_index("s") * NUM_CORES + lax.axis_index("c")
        b0 = wid * CHUNK
        pltpu.sync_copy(idx_hbm.at[:, pl.ds(b0, CHUNK)], idx_v)

        shr2 = lax.iota(jnp.int32, 16) >> 2
        n_full = vocab // STRIP
        tail = vocab - n_full * STRIP

        def transpose_strip(v0, sb):
            def j_body(j, carry2):
                rowv = shr2 + 4 * j
                rotv = ((v0 >> 2) + 4 * j + shr2) & 15
                for d in range(EMBED_DIM):
                    colv = ((lax.iota(jnp.int32, 16) & 3) * EMBED_DIM + d
                            + rotv) & 127
                    vals = tin_v[sb, d, pl.ds(j * 16, 16)]
                    plsc.store_scatter(tout_v, [rowv, colv], vals)
                return carry2

            lax.fori_loop(0, STRIP // 16, j_body, 0)
            m0 = pl.multiple_of(v0 >> 2, STRIP // ROWS_PER_MEGA)
            pltpu.sync_copy(tout_v, tscr_hbm.at[pl.ds(m0, STRIP // ROWS_PER_MEGA)])

        n_own = jnp.maximum(
            0, jnp.minimum(strips_per_w, n_full - wid * strips_per_w)
        )

        def fire_strip(k, sb):
            v0 = pl.multiple_of((wid * strips_per_w + k) * STRIP, STRIP)
            pltpu.async_copy(
                table_hbm.at[:, pl.ds(v0, STRIP)], tin_v.at[sb], ssem
            )
            return v0

        @pl.when(n_own > 0)
        def _():
            fire_strip(0, 0)

        def strip_body(k, carry):
            sb = lax.rem(k, 2)

            @pl.when(k + 1 < n_own)
            def _():
                fire_strip(k + 1, 1 - sb)

            v0 = pl.multiple_of((wid * strips_per_w + k) * STRIP, STRIP)
            pltpu.make_async_copy(
                table_hbm.at[:, pl.ds(v0, STRIP)], tin_v.at[sb], ssem
            ).wait()
            transpose_strip(v0, sb)
            return carry

        lax.fori_loop(0, n_own, strip_body, 0)
        if tail:
            @pl.when(wid == NUM_WORKERS - 1)
            def _():
                pltpu.sync_copy(
                    tail_hbm,
                    tscr_hbm.at[
                        pl.ds(n_full * (STRIP // ROWS_PER_MEGA),
                              tail // ROWS_PER_MEGA)
                    ],
                )
        plsc.subcore_barrier()
        pltpu.core_barrier(bsem, core_axis_name="c")
        plsc.subcore_barrier()

        def fire(h, buf):
            for g in range(CHUNK // 16):
                sl = pl.ds(g * 16, 16)
                mega_v[buf, sl] = lax.shift_right_logical(idx_v[h, sl], 2)
            return pltpu.async_copy(
                tscr_hbm.at[mega_v.at[buf]], rows_v.at[buf], gsem
            )

        fire(0, 0).wait()

        def chunk_body(h, carry):
            buf = lax.rem(h, 2)
            @pl.when(h + 1 < n_h)
            def _():
                fire(h + 1, 1 - buf)

            for g in range(CHUNK // 16):
                sl = pl.ds(g * 16, 16)
                iv = idx_v[h, sl]
                rem_v[sl] = (iv & 3) * EMBED_DIM + (lax.shift_right_logical(iv, 2) & 15)

            @pl.when(h >= 2)
            def _():
                pltpu.make_async_copy(
                    outt_v.at[buf], out_hbm.at[h - 2, :, pl.ds(b0, CHUNK)], osem
                ).wait()

            for g in range(CHUNK // 16):
                sl = pl.ds(g * 16, 16)
                rowv = lax.iota(jnp.int32, 16) + g * 16
                remg = rem_v[sl]
                for d in range(EMBED_DIM):
                    outt_v[buf, d, sl] = plsc.load_gather(
                        rows_v.at[buf], [rowv, (remg + d) & 127]
                    )

            pltpu.async_copy(
                outt_v.at[buf], out_hbm.at[h, :, pl.ds(b0, CHUNK)], osem
            )

            @pl.when(h + 1 < n_h)
            def _():
                pltpu.make_async_copy(
                    tscr_hbm.at[mega_v.at[1 - buf]], rows_v.at[1 - buf], gsem
                ).wait()

            return carry

        lax.fori_loop(0, n_h, chunk_body, 0)
        for h in (n_h - 2, n_h - 1):
            pltpu.make_async_copy(
                outt_v.at[h % 2], out_hbm.at[h, :, pl.ds(b0, CHUNK)], osem
            ).wait()

    return body


def kernel(inputs, table):
    b, h = inputs.shape
    vocab = table.shape[0]
    idx_t = inputs.astype(jnp.int32).T
    table_t = table.T
    n_full = vocab // STRIP
    tail = vocab - n_full * STRIP
    if tail:
        tail_mega = table[n_full * STRIP:].reshape(tail // ROWS_PER_MEGA, 128)
        n_tail = tail // ROWS_PER_MEGA
        rots = jnp.arange(n_tail, dtype=jnp.int32) & 15
        src = (jnp.arange(128, dtype=jnp.int32)[None, :] - rots[:, None]) % 128
        tail_mega = jnp.take_along_axis(tail_mega, src, axis=1)
    else:
        tail_mega = jnp.zeros((8, 128), jnp.float32)
    out_phys, _ = _make_kernel(h, b, vocab)(idx_t, table_t, tail_mega)
    return jnp.transpose(out_phys, (2, 0, 1))


# speedup vs baseline: 2.4333x; 1.0002x over previous
import functools

import jax
import jax.numpy as jnp
from jax import lax
from jax.experimental import pallas as pl
from jax.experimental.pallas import tpu as pltpu
from jax.experimental.pallas import tpu_sc as plsc

EMBED_DIM = 32
ROWS_PER_MEGA = 128 // EMBED_DIM
NUM_CORES = 2
NUM_SUBCORES = 16
NUM_WORKERS = NUM_CORES * NUM_SUBCORES
CHUNK = 128
STRIP = 512


@functools.lru_cache(maxsize=None)
def _make_kernel(n_h: int, n_b: int, vocab: int):
    n_mega = vocab // ROWS_PER_MEGA
    n_strips = -(-vocab // STRIP)
    strips_per_w = -(-n_strips // NUM_WORKERS)
    mesh = plsc.VectorSubcoreMesh(
        core_axis_name="c",
        subcore_axis_name="s",
        num_cores=NUM_CORES,
        num_subcores=NUM_SUBCORES,
    )

    @functools.partial(
        pl.kernel,
        out_type=(
            jax.ShapeDtypeStruct((n_h, EMBED_DIM, n_b), jnp.float32),
            jax.ShapeDtypeStruct((n_mega, 128), jnp.float32),
        ),
        mesh=mesh,
        scratch_types=[
            pltpu.VMEM((2, EMBED_DIM, STRIP), jnp.float32),
            pltpu.VMEM((STRIP // ROWS_PER_MEGA, 128), jnp.float32),
            pltpu.VMEM((n_h, CHUNK), jnp.int32),
            pltpu.VMEM((2, CHUNK), jnp.int32),
            pltpu.VMEM((CHUNK,), jnp.int32),
            pltpu.VMEM((2, CHUNK, 128), jnp.float32),
            pltpu.VMEM((2, EMBED_DIM, CHUNK), jnp.float32),
            pltpu.SemaphoreType.DMA,
            pltpu.SemaphoreType.DMA,
            pltpu.SemaphoreType.DMA,
            pltpu.SemaphoreType.REGULAR,
        ],
        compiler_params=pltpu.CompilerParams(needs_layout_passes=False),
    )
    def body(idx_hbm, table_hbm, tail_hbm, out_hbm, tscr_hbm, tin_v, tout_v,
             idx_v, mega_v, rem_v, rows_v, outt_v, gsem, ssem, osem, bsem):
        wid = lax.axis_index("s") * NUM_CORES + lax.axis_index("c")
        b0 = wid * CHUNK
        pltpu.sync_copy(idx_hbm.at[:, pl.ds(b0, CHUNK)], idx_v)

        shr2 = lax.iota(jnp.int32, 16) >> 2
        n_full = vocab // STRIP
        tail = vocab - n_full * STRIP

        def transpose_strip(v0, sb):
            def j_body(j, carry2):
                rowv = shr2 + 4 * j
                rotv = (((v0 >> 2) + 4 * j + shr2) & 15) * 4
                for d in range(EMBED_DIM):
                    colv = ((lax.iota(jnp.int32, 16) & 3) + 4 * d + rotv) & 127
                    vals = tin_v[sb, d, pl.ds(j * 16, 16)]
                    plsc.store_scatter(tout_v, [rowv, colv], vals)
                return carry2

            lax.fori_loop(0, STRIP // 16, j_body, 0)
            m0 = pl.multiple_of(v0 >> 2, STRIP // ROWS_PER_MEGA)
            pltpu.sync_copy(tout_v, tscr_hbm.at[pl.ds(m0, STRIP // ROWS_PER_MEGA)])

        n_own = jnp.maximum(
            0, jnp.minimum(strips_per_w, n_full - wid * strips_per_w)
        )

        def fire_strip(k, sb):
            v0 = pl.multiple_of((wid * strips_per_w + k) * STRIP, STRIP)
            pltpu.async_copy(
                table_hbm.at[:, pl.ds(v0, STRIP)], tin_v.at[sb], ssem
            )
            return v0

        @pl.when(n_own > 0)
        def _():
            fire_strip(0, 0)

        def strip_body(k, carry):
            sb = lax.rem(k, 2)

            @pl.when(k + 1 < n_own)
            def _():
                fire_strip(k + 1, 1 - sb)

            v0 = pl.multiple_of((wid * strips_per_w + k) * STRIP, STRIP)
            pltpu.make_async_copy(
                table_hbm.at[:, pl.ds(v0, STRIP)], tin_v.at[sb], ssem
            ).wait()
            transpose_strip(v0, sb)
            return carry

        lax.fori_loop(0, n_own, strip_body, 0)
        if tail:
            @pl.when(wid == NUM_WORKERS - 1)
            def _():
                pltpu.sync_copy(
                    tail_hbm,
                    tscr_hbm.at[
                        pl.ds(n_full * (STRIP // ROWS_PER_MEGA),
                              tail // ROWS_PER_MEGA)
                    ],
                )
        plsc.subcore_barrier()
        pltpu.core_barrier(bsem, core_axis_name="c")
        plsc.subcore_barrier()

        def fire(h, buf):
            for g in range(CHUNK // 16):
                sl = pl.ds(g * 16, 16)
                mega_v[buf, sl] = lax.shift_right_logical(idx_v[h, sl], 2)
            return pltpu.async_copy(
                tscr_hbm.at[mega_v.at[buf]], rows_v.at[buf], gsem
            )

        fire(0, 0).wait()

        def chunk_body(h, carry):
            buf = lax.rem(h, 2)
            @pl.when(h + 1 < n_h)
            def _():
                fire(h + 1, 1 - buf)

            for g in range(CHUNK // 16):
                sl = pl.ds(g * 16, 16)
                iv = idx_v[h, sl]
                rem_v[sl] = (iv & 3) + (lax.shift_right_logical(iv, 2) & 15) * 4

            @pl.when(h >= 2)
            def _():
                pltpu.make_async_copy(
                    outt_v.at[buf], out_hbm.at[h - 2, :, pl.ds(b0, CHUNK)], osem
                ).wait()

            for g in range(CHUNK // 16):
                sl = pl.ds(g * 16, 16)
                rowv = lax.iota(jnp.int32, 16) + g * 16
                remg = rem_v[sl]
                for d in range(EMBED_DIM):
                    outt_v[buf, d, sl] = plsc.load_gather(
                        rows_v.at[buf], [rowv, (remg + 4 * d) & 127]
                    )

            pltpu.async_copy(
                outt_v.at[buf], out_hbm.at[h, :, pl.ds(b0, CHUNK)], osem
            )

            @pl.when(h + 1 < n_h)
            def _():
                pltpu.make_async_copy(
                    tscr_hbm.at[mega_v.at[1 - buf]], rows_v.at[1 - buf], gsem
                ).wait()

            return carry

        lax.fori_loop(0, n_h, chunk_body, 0)
        for h in (n_h - 2, n_h - 1):
            pltpu.make_async_copy(
                outt_v.at[h % 2], out_hbm.at[h, :, pl.ds(b0, CHUNK)], osem
            ).wait()

    return body


def kernel(inputs, table):
    b, h = inputs.shape
    vocab = table.shape[0]
    idx_t = inputs.astype(jnp.int32).T
    table_t = table.T
    n_full = vocab // STRIP
    tail = vocab - n_full * STRIP
    if tail:
        n_tail = tail // ROWS_PER_MEGA
        tail_mega = (
            table[n_full * STRIP:]
            .reshape(n_tail, ROWS_PER_MEGA, EMBED_DIM)
            .transpose(0, 2, 1)
            .reshape(n_tail, 128)
        )
        rots = (jnp.arange(n_tail, dtype=jnp.int32) & 15) * 4
        src = (jnp.arange(128, dtype=jnp.int32)[None, :] - rots[:, None]) % 128
        tail_mega = jnp.take_along_axis(tail_mega, src, axis=1)
    else:
        tail_mega = jnp.zeros((8, 128), jnp.float32)
    out_phys, _ = _make_kernel(h, b, vocab)(idx_t, table_t, tail_mega)
    return jnp.transpose(out_phys, (2, 0, 1))
